# baseline (speedup 1.0000x reference)
import functools

import jax
import jax.numpy as jnp
from jax import lax
from jax.experimental import pallas as pl
from jax.experimental.pallas import tpu as pltpu
from jax.experimental.pallas import tpu_sc as plsc

NC = 2
NS = 16
L = 16
NW = NC * NS
NBUF = 4
OB = 8
SPLIT = 4096
BT = 256


def _lut_sc(x, tab_flat, a_t, b_t):
    B, F = x.shape
    TR, D = tab_flat.shape
    C, T = a_t.shape
    R = TR // T
    b_per_w = SPLIT // NW
    CHUNK = 16
    NCH = b_per_w // CHUNK

    mesh = plsc.VectorSubcoreMesh(
        core_axis_name="c", subcore_axis_name="s", num_cores=NC,
        num_subcores=NS)

    @functools.partial(
        pl.kernel,
        mesh=mesh,
        compiler_params=pltpu.CompilerParams(
            use_tc_tiling_on_sc=False, needs_layout_passes=False),
        out_type=jax.ShapeDtypeStruct((B, D), jnp.float32),
        scratch_types=[
            pltpu.VMEM((2, CHUNK, F), jnp.float32),
            pltpu.VMEM((NBUF, T, D), jnp.float32),
            pltpu.VMEM((2, OB, D), jnp.float32),
            pltpu.VMEM((C, T), jnp.int32),
            pltpu.VMEM((C, T), jnp.int32),
            pltpu.SemaphoreType.DMA,
            pltpu.SemaphoreType.DMA,
            pltpu.SemaphoreType.DMA,
        ],
    )
    def k(x_hbm, tab_hbm, a_hbm, b_hbm, out_hbm,
          xc, rows, orow, a_v, b_v, xsem, gsem, osem):
        cid = lax.axis_index("c")
        sid = lax.axis_index("s")
        wid = sid * NC + cid
        base = wid * b_per_w

        pltpu.sync_copy(a_hbm, a_v)
        pltpu.sync_copy(b_hbm, b_v)
        toff = lax.iota(jnp.int32, L) * R

        def fire_gather(tok):
            ch = lax.div(tok, CHUNK)
            csel = jnp.full((L,), lax.rem(ch, 2), dtype=jnp.int32)
            rsel = jnp.full((L,), lax.rem(tok, CHUNK), dtype=jnp.int32)
            idx = toff
            for c in range(C):
                av = plsc.load_gather(xc, [csel, rsel, a_v[c, :]])
                bv = plsc.load_gather(xc, [csel, rsel, b_v[c, :]])
                idx = idx | jnp.where(av > bv, jnp.int32(1 << c),
                                      jnp.int32(0))
            pltpu.async_copy(
                tab_hbm.at[idx], rows.at[lax.rem(tok, NBUF)], gsem)

        pltpu.async_copy(x_hbm.at[pl.ds(base, CHUNK)], xc.at[0], xsem)
        pltpu.make_async_copy(
            x_hbm.at[pl.ds(0, CHUNK)], xc.at[0], xsem).wait()
        pltpu.async_copy(
            x_hbm.at[pl.ds(base + CHUNK, CHUNK)], xc.at[1], xsem)
        for t in range(NBUF - 1):
            fire_gather(t)

        def tok_body(j, _):
            buf = lax.rem(j, NBUF)

            @pl.when(j + NBUF - 1 < b_per_w)
            def _():
                nxt = j + NBUF - 1

                @pl.when(lax.rem(nxt, CHUNK) == 0)
                def _():
                    nch = lax.div(nxt, CHUNK)
                    pltpu.make_async_copy(
                        x_hbm.at[pl.ds(0, CHUNK)], xc.at[lax.rem(nch, 2)],
                        xsem).wait()

                    @pl.when(nch + 1 < NCH)
                    def _():
                        pltpu.async_copy(
                            x_hbm.at[pl.ds(base + (nch + 1) * CHUNK, CHUNK)],
                            xc.at[lax.rem(nch + 1, 2)], xsem)

                fire_gather(nxt)

            pltpu.make_async_copy(
                tab_hbm.at[pl.ds(0, T)], rows.at[buf], gsem).wait()

            ob = lax.rem(lax.div(j, OB), 2)
            jo = lax.rem(j, OB)

            @pl.when((jo == 0) & (j >= 2 * OB))
            def _():
                pltpu.make_async_copy(
                    out_hbm.at[pl.ds(0, OB)], orow.at[ob], osem).wait()

            G = L

            def red_body(f, _):
                sl = pl.ds(f * G, G)
                vs = [rows[buf, r, sl] for r in range(T)]
                while len(vs) > 1:
                    vs = [vs[i] + vs[i + 1] for i in range(0, len(vs), 2)]
                orow[ob, jo, sl] = vs[0]
                return 0

            lax.fori_loop(0, D // G, red_body, 0, unroll=8)

            @pl.when(jo == OB - 1)
            def _():
                pltpu.async_copy(
                    orow.at[ob],
                    out_hbm.at[pl.ds(base + (lax.div(j, OB)) * OB, OB)], osem)
            return 0

        lax.fori_loop(0, b_per_w, tok_body, 0)
        pltpu.make_async_copy(out_hbm.at[pl.ds(0, OB)], orow.at[0], osem).wait()
        pltpu.make_async_copy(out_hbm.at[pl.ds(0, OB)], orow.at[1], osem).wait()

    return k(x, tab_flat, a_t, b_t)


def _tc_body(x_ref, tab_ref, a_ref, b_ref, y_ref, o_ref):
    T, R, D = tab_ref.shape
    BTk, F = x_ref.shape
    CT = a_ref.shape[1]
    C = CT // T

    xb = x_ref[...]
    ia = lax.broadcasted_iota(jnp.int32, (F, CT), 0)
    sel_a = (ia == a_ref[...]).astype(jnp.float32)
    sel_b = (ia == b_ref[...]).astype(jnp.float32)
    av = jnp.dot(xb, sel_a, precision=lax.Precision.HIGHEST)
    bv = jnp.dot(xb, sel_b, precision=lax.Precision.HIGHEST)
    gt = av > bv

    idx = jnp.zeros((BTk, T), jnp.int32)
    for c in range(C):
        idx = idx + jnp.where(gt[:, c * T:(c + 1) * T], jnp.int32(1 << c),
                              jnp.int32(0))

    iota_r = lax.broadcasted_iota(jnp.int32, (BTk, R), 1)
    acc = jnp.zeros((BTk, D), jnp.float32)
    for t in range(T):
        oh = (iota_r == idx[:, t][:, None]).astype(jnp.bfloat16)
        acc = acc + jnp.dot(oh, tab_ref[t],
                            preferred_element_type=jnp.float32)
    o_ref[...] = acc


def _lut_tc(x, tab_bf, a_row, b_row, y_base):
    B, F = x.shape
    Btc = B - SPLIT
    T, R, D = tab_bf.shape
    CT = a_row.shape[1]
    blk0 = SPLIT // BT
    return pl.pallas_call(
        _tc_body,
        grid=(Btc // BT,),
        in_specs=[
            pl.BlockSpec((BT, F), lambda i: (i + blk0, 0)),
            pl.BlockSpec((T, R, D), lambda i: (0, 0, 0)),
            pl.BlockSpec((1, CT), lambda i: (0, 0)),
            pl.BlockSpec((1, CT), lambda i: (0, 0)),
            pl.BlockSpec(memory_space=pl.ANY),
        ],
        out_specs=pl.BlockSpec((BT, D), lambda i: (i + blk0, 0)),
        out_shape=jax.ShapeDtypeStruct((B, D), jnp.float32),
        input_output_aliases={4: 0},
    )(x, tab_bf, a_row, b_row, y_base)


@jax.jit
def _lut(x, table, anchors_a, anchors_b):
    T, R, D = table.shape
    tab_flat = table.reshape(T * R, D)
    a_t = anchors_a.T.astype(jnp.int32)
    b_t = anchors_b.T.astype(jnp.int32)
    tab_bf = table.astype(jnp.bfloat16)
    a_row = a_t.reshape(1, -1)
    b_row = b_t.reshape(1, -1)
    y_sc = _lut_sc(x, tab_flat, a_t, b_t)
    return _lut_tc(x, tab_bf, a_row, b_row, y_sc)


def kernel(x, table, anchors_a, anchors_b):
    return _lut(x, table, anchors_a, anchors_b)


# speedup vs baseline: 1.3080x; 1.3080x over previous
import functools

import jax
import jax.numpy as jnp
from jax import lax
from jax.experimental import pallas as pl
from jax.experimental.pallas import tpu as pltpu
from jax.experimental.pallas import tpu_sc as plsc

NC = 2
NS = 16
L = 16
NW = NC * NS
NBUF = 4
OB = 8
SPLIT = 4096
BT = 512


def _lut_sc(x, tab_flat, a_t, b_t):
    B, F = x.shape
    TR, D = tab_flat.shape
    C, T = a_t.shape
    R = TR // T
    b_per_w = SPLIT // NW
    CHUNK = 16
    NCH = b_per_w // CHUNK

    mesh = plsc.VectorSubcoreMesh(
        core_axis_name="c", subcore_axis_name="s", num_cores=NC,
        num_subcores=NS)

    @functools.partial(
        pl.kernel,
        mesh=mesh,
        compiler_params=pltpu.CompilerParams(
            use_tc_tiling_on_sc=False, needs_layout_passes=False),
        out_type=jax.ShapeDtypeStruct((SPLIT, D), jnp.float32),
        scratch_types=[
            pltpu.VMEM((2, CHUNK, F), jnp.float32),
            pltpu.VMEM((NBUF, T, D), jnp.float32),
            pltpu.VMEM((2, OB, D), jnp.float32),
            pltpu.VMEM((C, T), jnp.int32),
            pltpu.VMEM((C, T), jnp.int32),
            pltpu.SemaphoreType.DMA,
            pltpu.SemaphoreType.DMA,
            pltpu.SemaphoreType.DMA,
        ],
    )
    def k(x_hbm, tab_hbm, a_hbm, b_hbm, out_hbm,
          xc, rows, orow, a_v, b_v, xsem, gsem, osem):
        cid = lax.axis_index("c")
        sid = lax.axis_index("s")
        wid = sid * NC + cid
        base = wid * b_per_w

        pltpu.sync_copy(a_hbm, a_v)
        pltpu.sync_copy(b_hbm, b_v)
        toff = lax.iota(jnp.int32, L) * R

        def fire_gather(tok):
            ch = lax.div(tok, CHUNK)
            csel = jnp.full((L,), lax.rem(ch, 2), dtype=jnp.int32)
            rsel = jnp.full((L,), lax.rem(tok, CHUNK), dtype=jnp.int32)
            idx = toff
            for c in range(C):
                av = plsc.load_gather(xc, [csel, rsel, a_v[c, :]])
                bv = plsc.load_gather(xc, [csel, rsel, b_v[c, :]])
                idx = idx | jnp.where(av > bv, jnp.int32(1 << c),
                                      jnp.int32(0))
            pltpu.async_copy(
                tab_hbm.at[idx], rows.at[lax.rem(tok, NBUF)], gsem)

        pltpu.async_copy(x_hbm.at[pl.ds(base, CHUNK)], xc.at[0], xsem)
        pltpu.make_async_copy(
            x_hbm.at[pl.ds(0, CHUNK)], xc.at[0], xsem).wait()
        pltpu.async_copy(
            x_hbm.at[pl.ds(base + CHUNK, CHUNK)], xc.at[1], xsem)
        for t in range(NBUF - 1):
            fire_gather(t)

        def tok_body(j, _):
            buf = lax.rem(j, NBUF)

            @pl.when(j + NBUF - 1 < b_per_w)
            def _():
                nxt = j + NBUF - 1

                @pl.when(lax.rem(nxt, CHUNK) == 0)
                def _():
                    nch = lax.div(nxt, CHUNK)
                    pltpu.make_async_copy(
                        x_hbm.at[pl.ds(0, CHUNK)], xc.at[lax.rem(nch, 2)],
                        xsem).wait()

                    @pl.when(nch + 1 < NCH)
                    def _():
                        pltpu.async_copy(
                            x_hbm.at[pl.ds(base + (nch + 1) * CHUNK, CHUNK)],
                            xc.at[lax.rem(nch + 1, 2)], xsem)

                fire_gather(nxt)

            pltpu.make_async_copy(
                tab_hbm.at[pl.ds(0, T)], rows.at[buf], gsem).wait()

            ob = lax.rem(lax.div(j, OB), 2)
            jo = lax.rem(j, OB)

            @pl.when((jo == 0) & (j >= 2 * OB))
            def _():
                pltpu.make_async_copy(
                    out_hbm.at[pl.ds(0, OB)], orow.at[ob], osem).wait()

            G = L

            def red_body(f, _):
                sl = pl.ds(f * G, G)
                vs = [rows[buf, r, sl] for r in range(T)]
                while len(vs) > 1:
                    vs = [vs[i] + vs[i + 1] for i in range(0, len(vs), 2)]
                orow[ob, jo, sl] = vs[0]
                return 0

            lax.fori_loop(0, D // G, red_body, 0, unroll=8)

            @pl.when(jo == OB - 1)
            def _():
                pltpu.async_copy(
                    orow.at[ob],
                    out_hbm.at[pl.ds(base + (lax.div(j, OB)) * OB, OB)], osem)
            return 0

        lax.fori_loop(0, b_per_w, tok_body, 0)
        pltpu.make_async_copy(out_hbm.at[pl.ds(0, OB)], orow.at[0], osem).wait()
        pltpu.make_async_copy(out_hbm.at[pl.ds(0, OB)], orow.at[1], osem).wait()

    return k(x, tab_flat, a_t, b_t)


def _tc_body(x_ref, tab_ref, a_ref, b_ref, o_ref):
    T, R, D = tab_ref.shape
    BTk, F = x_ref.shape
    CT = a_ref.shape[1]
    C = CT // T

    xb = x_ref[...]
    ia = lax.broadcasted_iota(jnp.int32, (F, CT), 0)
    sel = ((ia == a_ref[...]).astype(jnp.float32)
           - (ia == b_ref[...]).astype(jnp.float32))
    diff = jnp.dot(xb, sel, precision=lax.Precision.HIGHEST)
    gt = diff > 0

    idx = jnp.zeros((BTk, T), jnp.int32)
    for c in range(C):
        idx = idx + jnp.where(gt[:, c * T:(c + 1) * T], jnp.int32(1 << c),
                              jnp.int32(0))

    iota_r = lax.broadcasted_iota(jnp.int32, (BTk, R), 1)
    acc = jnp.zeros((BTk, D), jnp.float32)
    for t in range(T):
        oh = (iota_r == idx[:, t][:, None]).astype(jnp.bfloat16)
        acc = acc + jnp.dot(oh, tab_ref[t],
                            preferred_element_type=jnp.float32)
    o_ref[...] = acc


def _lut_tc(x_tc, tab_bf, a_row, b_row):
    Btc, F = x_tc.shape
    T, R, D = tab_bf.shape
    CT = a_row.shape[1]
    return pl.pallas_call(
        _tc_body,
        grid=(Btc // BT,),
        in_specs=[
            pl.BlockSpec((BT, F), lambda i: (i, 0)),
            pl.BlockSpec((T, R, D), lambda i: (0, 0, 0)),
            pl.BlockSpec((1, CT), lambda i: (0, 0)),
            pl.BlockSpec((1, CT), lambda i: (0, 0)),
        ],
        out_specs=pl.BlockSpec((BT, D), lambda i: (i, 0)),
        out_shape=jax.ShapeDtypeStruct((Btc, D), jnp.float32),
    )(x_tc, tab_bf, a_row, b_row)


@jax.jit
def _lut(x, table, anchors_a, anchors_b):
    T, R, D = table.shape
    tab_flat = table.reshape(T * R, D)
    a_t = anchors_a.T.astype(jnp.int32)
    b_t = anchors_b.T.astype(jnp.int32)
    tab_bf = table.astype(jnp.bfloat16)
    a_row = a_t.reshape(1, -1)
    b_row = b_t.reshape(1, -1)
    y_sc = _lut_sc(x[:SPLIT], tab_flat, a_t, b_t)
    y_tc = _lut_tc(x[SPLIT:], tab_bf, a_row, b_row)
    return jnp.concatenate([y_sc, y_tc], axis=0)


def kernel(x, table, anchors_a, anchors_b):
    return _lut(x, table, anchors_a, anchors_b)
